# Initial kernel scaffold; baseline (speedup 1.0000x reference)
import jax
import jax.numpy as jnp
from jax.experimental import pallas as pl


def kernel(x, segment_info, table, seg_table, pos):
    raise NotImplementedError("write your pallas kernel here")


# speedup vs baseline: 1.2698x; 1.2698x over previous
import functools

import jax
import jax.numpy as jnp
from jax import lax
from jax.experimental import pallas as pl
from jax.experimental.pallas import tpu as pltpu
from jax.experimental.pallas import tpu_sc as plsc

B, L, D = 1024, 200, 64
NC, NS = 2, 16
NW = NC * NS
TOK = B * L
PER_W = TOK // NW
CHUNK = 128
NCH = PER_W // CHUNK
NBUF = 5
NGROUP = NCH // NBUF


def _comb_body(seg_ref, pos_ref, out_ref):
    p = pos_ref[...]
    for s in range(3):
        out_ref[s, :, :] = p + seg_ref[s, :][None, :]


def _make_comb(seg_table, pos_l):
    return pl.pallas_call(
        _comb_body,
        out_shape=jax.ShapeDtypeStruct((3, L, D), jnp.float32),
    )(seg_table, pos_l)


def _sc_body(x3, seg3, table, comb, out, xi_v, ci_v, buf_a, buf_b,
             sem_a, sem_b, sem_o):
    wid = lax.axis_index("s") * NC + lax.axis_index("c")
    base_row = wid * PER_W
    pltpu.sync_copy(x3.at[wid], xi_v)
    pltpu.sync_copy(seg3.at[wid], ci_v)

    iota16 = lax.iota(jnp.int32, 16)

    def build_row(r, carry):
        for k in range(CHUNK // 16):
            col = k * 16
            seg = ci_v[r, pl.ds(col, 16)]
            lpos = lax.rem(r * CHUNK + col + iota16, L)
            ci_v[r, pl.ds(col, 16)] = seg * L + lpos
        return carry

    lax.fori_loop(0, NCH, build_row, 0)

    def group(g, carry):
        for b in range(NBUF):
            c = g * NBUF + b

            @pl.when(g > 0)
            def _wait_prev():
                prev = base_row + lax.max(c - NBUF, 0) * CHUNK
                pltpu.make_async_copy(
                    buf_a.at[b], out.at[pl.ds(prev, CHUNK)], sem_o.at[b]
                ).wait()

            pltpu.make_async_copy(
                table.at[xi_v.at[c]], buf_a.at[b], sem_a.at[b]).start()
            pltpu.make_async_copy(
                comb.at[ci_v.at[c]], buf_b.at[b], sem_b.at[b]).start()

        for b in range(NBUF):
            c = g * NBUF + b
            pltpu.make_async_copy(
                table.at[xi_v.at[c]], buf_a.at[b], sem_a.at[b]).wait()
            pltpu.make_async_copy(
                comb.at[ci_v.at[c]], buf_b.at[b], sem_b.at[b]).wait()
            ba, bb = buf_a.at[b], buf_b.at[b]

            def add_tok(t, inner):
                for k in range(D // 16):
                    col = k * 16
                    ba[t, pl.ds(col, 16)] = (
                        ba[t, pl.ds(col, 16)] + bb[t, pl.ds(col, 16)])
                return inner

            lax.fori_loop(0, CHUNK, add_tok, 0)
            pltpu.make_async_copy(
                buf_a.at[b], out.at[pl.ds(base_row + c * CHUNK, CHUNK)],
                sem_o.at[b]).start()
        return carry

    lax.fori_loop(0, NGROUP, group, 0)
    for b in range(NBUF):
        c = (NGROUP - 1) * NBUF + b
        pltpu.make_async_copy(
            buf_a.at[b], out.at[pl.ds(base_row + c * CHUNK, CHUNK)],
            sem_o.at[b]).wait()


_sc_call = pl.kernel(
    _sc_body,
    out_type=jax.ShapeDtypeStruct((TOK, D), jnp.float32),
    mesh=plsc.VectorSubcoreMesh(
        core_axis_name="c", subcore_axis_name="s",
        num_cores=NC, num_subcores=NS),
    scratch_types=[
        pltpu.VMEM((NCH, CHUNK), jnp.int32),
        pltpu.VMEM((NCH, CHUNK), jnp.int32),
        pltpu.VMEM((NBUF, CHUNK, D), jnp.float32),
        pltpu.VMEM((NBUF, CHUNK, D), jnp.float32),
        pltpu.SemaphoreType.DMA((NBUF,)),
        pltpu.SemaphoreType.DMA((NBUF,)),
        pltpu.SemaphoreType.DMA((NBUF,)),
    ],
    compiler_params=pltpu.CompilerParams(use_tc_tiling_on_sc=False),
)


def kernel(x, segment_info, table, seg_table, pos):
    comb = _make_comb(seg_table, pos[:L]).reshape(3 * L, D)
    x3 = x.reshape(NW, NCH, CHUNK).astype(jnp.int32)
    s3 = segment_info.reshape(NW, NCH, CHUNK).astype(jnp.int32)
    out = _sc_call(x3, s3, table, comb)
    return out.reshape(B, L, D)
